# baseline (speedup 1.0000x reference)
import jax
import jax.numpy as jnp
from jax import lax
from jax.experimental import pallas as pl
from jax.experimental.pallas import tpu as pltpu
from jax.experimental.pallas import tpu_sc as plsc

_T = 8192
_B = 4
_C = 1024
_M = 2 * _T - 1
_NC = 2
_NS = 16
_NW = _NC * _NS
_TW = _T // _NW
_K = 4
_NCHUNK = _TW // _K
_L = 16
_NPAIR = _NCHUNK // 2


def _sc_body(y_hbm, out_hbm, vb0, vb1, ob0, ob1, ls0, ls1, ss0, ss1):
    wid = lax.axis_index("s") * _NC + lax.axis_index("c")
    base = wid * _TW
    vbufs = (vb0, vb1)
    obufs = (ob0, ob1)
    lsems = (ls0, ls1)
    ssems = (ss0, ss1)

    def issue_load(ci, b):
        m0 = base + ci * _K
        interior = jnp.logical_and(m0 > 0, m0 + _K < _T)

        @pl.when(interior)
        def _():
            pltpu.async_copy(y_hbm.at[pl.ds(m0 - 1, _K + 2)],
                             vbufs[b], lsems[b])

        @pl.when(jnp.logical_not(interior))
        def _():
            prow = jnp.maximum(m0 - 1, 0)
            nrow = jnp.minimum(m0 + _K, _T - 1)
            pltpu.async_copy(y_hbm.at[pl.ds(prow, 1)],
                             vbufs[b].at[pl.ds(0, 1)], lsems[b])
            pltpu.async_copy(y_hbm.at[pl.ds(m0, _K)],
                             vbufs[b].at[pl.ds(1, _K)], lsems[b])
            pltpu.async_copy(y_hbm.at[pl.ds(nrow, 1)],
                             vbufs[b].at[pl.ds(_K + 1, 1)], lsems[b])

    def wait_load(b):
        pltpu.make_async_copy(y_hbm.at[pl.ds(0, _K + 2)],
                              vbufs[b], lsems[b]).wait()

    def issue_store(ci, b):
        m0 = base + ci * _K
        pltpu.async_copy(obufs[b], out_hbm.at[pl.ds(2 * m0, 2 * _K)],
                         ssems[b])

    def wait_store(b):
        pltpu.make_async_copy(obufs[b], out_hbm.at[pl.ds(0, 2 * _K)],
                              ssems[b]).wait()

    def compute(ci, b):
        m0f = (base + ci * _K).astype(jnp.float32)
        avs = []
        bvs = []
        for i in range(_K):
            a = (m0f + i) * (1.0 / _M)
            bw = (m0f + (i + _T)) * (1.0 / _M)
            avs.append(jnp.broadcast_to(a, (_L,)))
            bvs.append(jnp.broadcast_to(bw, (_L,)))
        vb = vbufs[b]
        ob = obufs[b]

        @plsc.parallel_loop(0, _C, 2 * _L, unroll=1)
        def _(j):
            for s in range(_B):
                for u in (0, _L):
                    ju = j + u
                    lv = [vb[r, s, pl.ds(ju, _L)] for r in range(_K + 2)]
                    diff = [lv[r] - lv[r + 1] for r in range(_K + 1)]
                    for i in range(_K):
                        ob[2 * i, s, pl.ds(ju, _L)] = (
                            lv[i + 1] + avs[i] * diff[i])
                        ob[2 * i + 1, s, pl.ds(ju, _L)] = (
                            lv[i + 2] + bvs[i] * diff[i + 1])

    issue_load(0, 0)
    issue_load(1, 1)

    def pair_body(g, carry):
        for b in range(2):
            ci = 2 * g + b
            wait_load(b)

            @pl.when(g >= 1)
            def _():
                wait_store(b)

            compute(ci, b)
            issue_store(ci, b)

            @pl.when(g <= _NPAIR - 2)
            def _():
                issue_load(ci + 2, b)

        return carry

    lax.fori_loop(0, _NPAIR, pair_body, 0)
    wait_store(0)
    wait_store(1)


def kernel(y):
    mesh = plsc.VectorSubcoreMesh(core_axis_name="c", subcore_axis_name="s")
    return pl.kernel(
        _sc_body,
        mesh=mesh,
        out_type=jax.ShapeDtypeStruct((2 * _T, _B, _C), jnp.float32),
        scratch_types=[
            pltpu.VMEM((_K + 2, _B, _C), jnp.float32),
            pltpu.VMEM((_K + 2, _B, _C), jnp.float32),
            pltpu.VMEM((2 * _K, _B, _C), jnp.float32),
            pltpu.VMEM((2 * _K, _B, _C), jnp.float32),
            pltpu.SemaphoreType.DMA,
            pltpu.SemaphoreType.DMA,
            pltpu.SemaphoreType.DMA,
            pltpu.SemaphoreType.DMA,
        ],
    )(y)


# speedup vs baseline: 1.0752x; 1.0752x over previous
import jax
import jax.numpy as jnp
from jax import lax
from jax.experimental import pallas as pl
from jax.experimental.pallas import tpu as pltpu
from jax.experimental.pallas import tpu_sc as plsc

_T = 8192
_B = 4
_C = 1024
_M = 2 * _T - 1
_NC = 2
_NS = 16
_NW = _NC * _NS
_TW = _T // _NW
_W = 4
_NWIN = _TW // _W
_L = 16


def _sc_body(y_hbm, out_hbm, vb0, vb1, vb2, ob0, ob1,
             ls0, ls1, ls2, ss0, ss1):
    wid = lax.axis_index("s") * _NC + lax.axis_index("c")
    base = wid * _TW
    vbufs = (vb0, vb1, vb2)
    obufs = (ob0, ob1)
    lsems = (ls0, ls1, ls2)
    ssems = (ss0, ss1)

    def issue_load(k, slot):
        start = jnp.minimum(base + k * _W, _T - _W)
        pltpu.async_copy(y_hbm.at[pl.ds(start, _W)], vbufs[slot],
                         lsems[slot])

    def wait_load(slot):
        pltpu.make_async_copy(y_hbm.at[pl.ds(0, _W)], vbufs[slot],
                              lsems[slot]).wait()

    def issue_store(ci, b):
        row = 2 * (base + ci * _W) + 1
        last_special = jnp.logical_and(wid == _NW - 1, ci == _NWIN - 1)

        @pl.when(last_special)
        def _():
            pltpu.async_copy(obufs[b].at[pl.ds(0, 2 * _W - 1)],
                             out_hbm.at[pl.ds(row, 2 * _W - 1)], ssems[b])

        @pl.when(jnp.logical_not(last_special))
        def _():
            pltpu.async_copy(obufs[b], out_hbm.at[pl.ds(row, 2 * _W)],
                             ssems[b])

    def wait_store(b):
        pltpu.make_async_copy(obufs[b], out_hbm.at[pl.ds(0, 2 * _W)],
                              ssems[b]).wait()

    def compute(ci, sa, sn, b):
        sf = (base + ci * _W).astype(jnp.float32)
        aws = []
        bws = []
        for i in range(_W):
            bw = (sf + (i + _T)) * (1.0 / _M)
            aw = (sf + (i + 1)) * (1.0 / _M)
            bws.append(jnp.broadcast_to(bw, (_L,)))
            aws.append(jnp.broadcast_to(aw, (_L,)))
        va = vbufs[sa]
        vn = vbufs[sn]
        ob = obufs[b]

        @plsc.parallel_loop(0, _C, 2 * _L, unroll=1)
        def _(j):
            for s in range(_B):
                for u in (0, _L):
                    ju = j + u
                    lv = [va[r, s, pl.ds(ju, _L)] for r in range(_W)]
                    lv.append(vn[0, s, pl.ds(ju, _L)])
                    diff = [lv[i] - lv[i + 1] for i in range(_W)]
                    for i in range(_W):
                        ob[2 * i, s, pl.ds(ju, _L)] = (
                            lv[i + 1] + bws[i] * diff[i])
                        ob[2 * i + 1, s, pl.ds(ju, _L)] = (
                            lv[i + 1] + aws[i] * diff[i])

    @pl.when(wid == 0)
    def _():
        pltpu.sync_copy(y_hbm.at[pl.ds(0, 1)], ob0.at[pl.ds(0, 1)])
        pltpu.sync_copy(ob0.at[pl.ds(0, 1)], out_hbm.at[pl.ds(0, 1)])

    issue_load(0, 0)
    issue_load(1, 1)
    issue_load(2, 2)
    wait_load(0)

    def chunk_step(ci, sa, sn, b):
        wait_load(sn)

        @pl.when(ci >= 2)
        def _():
            wait_store(b)

        compute(ci, sa, sn, b)
        issue_store(ci, b)

        @pl.when(ci + 3 <= _NWIN)
        def _():
            issue_load(ci + 3, sa)

    def six_body(g, carry):
        for ph in range(6):
            chunk_step(6 * g + ph, ph % 3, (ph + 1) % 3, ph % 2)
        return carry

    lax.fori_loop(0, _NWIN // 6, six_body, 0)
    for ph in range(_NWIN % 6):
        ci = (_NWIN // 6) * 6 + ph
        chunk_step(jnp.int32(ci), ci % 3, (ci + 1) % 3, ci % 2)

    wait_store(_NWIN % 2)

    last_b = (_NWIN - 1) % 2
    @pl.when(wid == _NW - 1)
    def _():
        pltpu.make_async_copy(obufs[last_b].at[pl.ds(0, 2 * _W - 1)],
                              out_hbm.at[pl.ds(0, 2 * _W - 1)],
                              ssems[last_b]).wait()

    @pl.when(wid != _NW - 1)
    def _():
        wait_store(last_b)


def kernel(y):
    mesh = plsc.VectorSubcoreMesh(core_axis_name="c", subcore_axis_name="s")
    return pl.kernel(
        _sc_body,
        mesh=mesh,
        out_type=jax.ShapeDtypeStruct((2 * _T, _B, _C), jnp.float32),
        scratch_types=[
            pltpu.VMEM((_W, _B, _C), jnp.float32),
            pltpu.VMEM((_W, _B, _C), jnp.float32),
            pltpu.VMEM((_W, _B, _C), jnp.float32),
            pltpu.VMEM((2 * _W, _B, _C), jnp.float32),
            pltpu.VMEM((2 * _W, _B, _C), jnp.float32),
            pltpu.SemaphoreType.DMA,
            pltpu.SemaphoreType.DMA,
            pltpu.SemaphoreType.DMA,
            pltpu.SemaphoreType.DMA,
            pltpu.SemaphoreType.DMA,
        ],
    )(y)
